# baseline (speedup 1.0000x reference)
import functools

import jax
import jax.numpy as jnp
import numpy as np
from jax import lax
from jax.experimental import pallas as pl
from jax.experimental.pallas import tpu as pltpu
from jax.experimental.pallas import tpu_sc as plsc

_N = 10000
_D = 128
_NC = 2
_NS = 16
_NW = _NC * _NS
_L = 16
_CH = 128
_SQ = 8
_NP = 10240
_NR = _NP // _NS

_SC_PARAMS = pltpu.CompilerParams(use_tc_tiling_on_sc=False,
                                  needs_layout_passes=False)


def _rsqrt_nr(v):
    i = plsc.bitcast(v, jnp.int32)
    y = plsc.bitcast(jnp.int32(0x5F3759DF) - lax.shift_right_logical(i, 1), jnp.float32)
    for _ in range(3):
        y = y * (1.5 - 0.5 * v * y * y)
    return y


def _scale_rows_packed(rows_pk, frows, w_b, j):
    def grp_body(g, _):
        wvec = w_b[j, pl.ds(g * _L, _L)]
        for r in range(_L):
            ws = wvec[r]
            i = g * _L + r
            for c in range(_D // (2 * _L)):
                v = rows_pk[i, pl.ds(c * _L, _L)]
                bb = plsc.bitcast(v, jnp.bfloat16)
                a, b = plsc.unpack(bb, format=plsc.PackFormat.INTERLEAVED)
                frows[i, pl.ds((2 * c) * _L, _L)] = a * ws
                frows[i, pl.ds((2 * c + 1) * _L, _L)] = b * ws
        return 0
    lax.fori_loop(0, _CH // _L, grp_body, 0)


_HH = _CH // 2


def _gather_start(h_hbm, sidx_b, rows_v, gsem, j, p):
    pltpu.async_copy(h_hbm.at[sidx_b.at[j]], rows_v.at[p], gsem.at[p])


def _gather_wait(h_hbm, sidx_b, rows_v, gsem, j, p):
    pltpu.make_async_copy(h_hbm.at[sidx_b.at[j]], rows_v.at[p],
                          gsem.at[p]).wait()


def _prop_pipelined(h_hbm, sidx_b, didx_b, w_b, rows_v, frows, acc_sh, gsem):
    _gather_start(h_hbm, sidx_b, rows_v, gsem, 0, 0)

    def chunk(j, _):
        p = j % 2
        _gather_wait(h_hbm, sidx_b, rows_v, gsem, j, p)

        @pl.when(j + 1 < _SQ)
        def _start_next_gather():
            _gather_start(h_hbm, sidx_b, rows_v, gsem, j + 1, 1 - p)

        _scale_rows_packed(rows_v.at[p], frows, w_b, j)
        pltpu.sync_copy(frows, acc_sh.at[didx_b.at[j]], add=True)
        return 0
    lax.fori_loop(0, _SQ, chunk, 0)


def _make_sc1(Q):
    mesh = plsc.VectorSubcoreMesh(core_axis_name="c", subcore_axis_name="s")

    @functools.partial(
        pl.kernel,
        out_type=(
            jax.ShapeDtypeStruct((_NW, Q, _SQ, _CH), jnp.float32),
            jax.ShapeDtypeStruct((_NC, _NP, _D), jnp.float32),
        ),
        mesh=mesh,
        compiler_params=_SC_PARAMS,
        scratch_types=[
            pltpu.VMEM_SHARED((_NP, _D), jnp.float32),
            pltpu.VMEM_SHARED((_NP,), jnp.float32),
            pltpu.VMEM_SHARED((_NP,), jnp.float32),
            pltpu.VMEM((_SQ, _CH), jnp.int32),
            pltpu.VMEM((_SQ, _CH), jnp.int32),
            pltpu.VMEM((_SQ, _CH), jnp.float32),
            pltpu.VMEM((_SQ, _CH), jnp.float32),
            pltpu.VMEM((_SQ, _CH), jnp.float32),
            pltpu.VMEM((_CH,), jnp.float32),
            pltpu.VMEM((_CH,), jnp.float32),
            pltpu.VMEM((_NR,), jnp.float32),
            pltpu.VMEM((2, _CH, _D // 2), jnp.float32),
            pltpu.VMEM((_CH, _D), jnp.float32),
            pltpu.SemaphoreType.DMA,
            pltpu.SemaphoreType.DMA((2,)),
        ],
    )
    def sc1(x_hbm, src4, dst4, ew4, zrows, zvec, w_out, p_out,
            acc_sh, deg_sh, dis_sh, sidx_b, didx_b, ew_b, w_b, ewm_b,
            dsv, ddv, disbuf, rows_v, frows, sem, gsem):
        sid = lax.axis_index("s")
        cid = lax.axis_index("c")
        wid = sid * _NC + cid
        r0 = sid * _NR

        pltpu.sync_copy(zrows, acc_sh.at[pl.ds(r0, _NR)])
        pltpu.sync_copy(zvec, deg_sh.at[pl.ds(r0, _NR)])
        plsc.subcore_barrier()

        for half in range(2):
            slc = sid + half * _NS

            def deg_sq(q, _):
                pltpu.sync_copy(src4.at[slc, q], sidx_b)
                pltpu.sync_copy(dst4.at[slc, q], didx_b)
                pltpu.sync_copy(ew4.at[slc, q], ew_b)

                def deg_chunk(j, __):
                    for g in range(_CH // _L):
                        sl16 = pl.ds(g * _L, _L)
                        s = sidx_b[j, sl16]
                        d = didx_b[j, sl16]
                        e = ew_b[j, sl16]
                        ewm_b[j, sl16] = jnp.where(s != d, e, 0.0)
                    pltpu.async_copy(ewm_b.at[j], deg_sh.at[sidx_b.at[j]],
                                     sem, add=True)
                    return 0
                lax.fori_loop(0, _SQ, deg_chunk, 0)

                def deg_drain(j, __):
                    pltpu.make_async_copy(ewm_b.at[j],
                                          deg_sh.at[sidx_b.at[j]], sem).wait()
                    return 0
                lax.fori_loop(0, _SQ, deg_drain, 0)
                return 0
            lax.fori_loop(0, Q, deg_sq, 0)
        plsc.subcore_barrier()

        pltpu.sync_copy(deg_sh.at[pl.ds(r0, _NR)], disbuf)

        def dis_body(k, _):
            acc = disbuf[pl.ds(k * _L, _L)]
            y = _rsqrt_nr(acc)
            disbuf[pl.ds(k * _L, _L)] = jnp.where(acc > 0.0, y, 0.0)
            return 0
        lax.fori_loop(0, _NR // _L, dis_body, 0)
        pltpu.sync_copy(disbuf, dis_sh.at[pl.ds(r0, _NR)])
        plsc.subcore_barrier()

        def wp_sq(q, _):
            pltpu.sync_copy(src4.at[wid, q], sidx_b)
            pltpu.sync_copy(dst4.at[wid, q], didx_b)
            pltpu.sync_copy(ew4.at[wid, q], ew_b)

            def w_chunk(j, __):
                cps = pltpu.async_copy(dis_sh.at[sidx_b.at[j]], dsv, sem)
                cpd = pltpu.async_copy(dis_sh.at[didx_b.at[j]], ddv, sem)
                cps.wait()
                cpd.wait()
                for g in range(_CH // _L):
                    sl16 = pl.ds(g * _L, _L)
                    s = sidx_b[j, sl16]
                    d = didx_b[j, sl16]
                    e = ew_b[j, sl16]
                    w_b[j, sl16] = jnp.where(s != d, (-dsv[sl16]) * e * ddv[sl16], 0.0)
                return 0
            lax.fori_loop(0, _SQ, w_chunk, 0)
            pltpu.sync_copy(w_b, w_out.at[wid, q])

            _prop_pipelined(x_hbm, sidx_b, didx_b, w_b, rows_v, frows,
                            acc_sh, gsem)
            return 0
        lax.fori_loop(0, Q, wp_sq, 0)
        plsc.subcore_barrier()

        pltpu.sync_copy(acc_sh.at[pl.ds(r0, _NR)], p_out.at[cid, pl.ds(r0, _NR)])

    return sc1


def _make_sc2(Q):
    mesh = plsc.VectorSubcoreMesh(core_axis_name="c", subcore_axis_name="s")

    @functools.partial(
        pl.kernel,
        out_type=jax.ShapeDtypeStruct((_NC, _NP, _D), jnp.float32),
        mesh=mesh,
        compiler_params=_SC_PARAMS,
        scratch_types=[
            pltpu.VMEM_SHARED((_NP, _D), jnp.float32),
            pltpu.VMEM((_SQ, _CH), jnp.int32),
            pltpu.VMEM((_SQ, _CH), jnp.int32),
            pltpu.VMEM((_SQ, _CH), jnp.float32),
            pltpu.VMEM((2, _CH, _D // 2), jnp.float32),
            pltpu.VMEM((_CH, _D), jnp.float32),
            pltpu.SemaphoreType.DMA((2,)),
        ],
    )
    def sc2(h_hbm, src4, dst4, w_hbm, zrows, p_out,
            acc_sh, sidx_b, didx_b, w_b, rows_v, frows, gsem):
        sid = lax.axis_index("s")
        cid = lax.axis_index("c")
        wid = sid * _NC + cid
        r0 = sid * _NR

        pltpu.sync_copy(zrows, acc_sh.at[pl.ds(r0, _NR)])
        plsc.subcore_barrier()

        def prop_sq(q, _):
            pltpu.sync_copy(src4.at[wid, q], sidx_b)
            pltpu.sync_copy(dst4.at[wid, q], didx_b)
            pltpu.sync_copy(w_hbm.at[wid, q], w_b)
            _prop_pipelined(h_hbm, sidx_b, didx_b, w_b, rows_v, frows,
                            acc_sh, gsem)
            return 0
        lax.fori_loop(0, Q, prop_sq, 0)
        plsc.subcore_barrier()
        pltpu.sync_copy(acc_sh.at[pl.ds(r0, _NR)], p_out.at[cid, pl.ds(r0, _NR)])

    return sc2


_RB = 1000
_OC = 384


def _tc_combine1(p1_ref, x_ref, a_ref, bias_ref, tx_ref, s_ref):
    tx_ref[...] = p1_ref[0] + p1_ref[1]
    s_ref[...] = (
        jnp.dot(x_ref[...], a_ref[...], preferred_element_type=jnp.float32)
        + bias_ref[...][0:1, :]
    )


def _tc_combine2(s_ref, tx_ref, p2_ref, b_ref, c_ref, o_ref):
    t2 = p2_ref[0] + p2_ref[1]
    res = (
        s_ref[...]
        + jnp.dot(tx_ref[...], b_ref[...], preferred_element_type=jnp.float32)
        + jnp.dot(t2, c_ref[...], preferred_element_type=jnp.float32)
    )
    o_ref[...] = res[:, :300]


def kernel(x, edge_index, edge_weight, W1_0, b1, W2_0, W2_1, b2, W3_0, W3_1, W3_2, b3):
    E = edge_index.shape[1]
    Q = -(-E // (_NW * _SQ * _CH))
    Ep = _NW * Q * _SQ * _CH

    src = edge_index[0]
    dst = edge_index[1]
    pad = Ep - E
    src4 = jnp.pad(src, (0, pad)).reshape(_NW, Q, _SQ, _CH)
    dst4 = jnp.pad(dst, (0, pad)).reshape(_NW, Q, _SQ, _CH)
    ew4 = jnp.pad(edge_weight, (0, pad)).reshape(_NW, Q, _SQ, _CH)
    zrows = jnp.zeros((_NR, _D), jnp.float32)
    zvec = jnp.zeros((_NR,), jnp.float32)

    xpk = lax.bitcast_convert_type(
        x.astype(jnp.bfloat16).reshape(_N, _D // 2, 2), jnp.float32)
    w_e, p1 = _make_sc1(Q)(xpk, src4, dst4, ew4, zrows, zvec)

    A = jnp.pad(jnp.concatenate([W1_0, W2_0, W3_0 - W3_2], axis=1),
                ((0, 0), (0, _OC - 300)))
    B = jnp.pad(jnp.concatenate([jnp.zeros_like(W2_1), W2_1, W3_1], axis=1),
                ((0, 0), (0, _OC - 300)))
    C = jnp.pad(jnp.concatenate([jnp.zeros_like(W3_2), jnp.zeros_like(W3_2),
                                 2.0 * W3_2], axis=1), ((0, 0), (0, _OC - 300)))
    bias = jnp.broadcast_to(
        jnp.pad(jnp.concatenate([b1, b2, b3]), (0, _OC - 300)), (8, _OC))

    perm = np.zeros((_D,), np.int32)
    for c_ in range(_D // 32):
        for k_ in range(16):
            perm[32 * c_ + k_] = 32 * c_ + 2 * k_
            perm[32 * c_ + 16 + k_] = 32 * c_ + 2 * k_ + 1
    perm2 = perm[perm]
    B = B[perm, :]
    C = C[perm2, :]

    grid = (_N // _RB,)
    tx1, s_acc = pl.pallas_call(
        _tc_combine1,
        grid=grid,
        in_specs=[
            pl.BlockSpec((_NC, _RB, _D), lambda i: (0, i, 0)),
            pl.BlockSpec((_RB, _D), lambda i: (i, 0)),
            pl.BlockSpec((_D, _OC), lambda i: (0, 0)),
            pl.BlockSpec((8, _OC), lambda i: (0, 0)),
        ],
        out_specs=[
            pl.BlockSpec((_RB, _D), lambda i: (i, 0)),
            pl.BlockSpec((_RB, _OC), lambda i: (i, 0)),
        ],
        out_shape=[
            jax.ShapeDtypeStruct((_N, _D), jnp.float32),
            jax.ShapeDtypeStruct((_N, _OC), jnp.float32),
        ],
    )(p1, x, A, bias)

    tx1pk = lax.bitcast_convert_type(
        tx1.astype(jnp.bfloat16).reshape(_N, _D // 2, 2), jnp.float32)
    p2 = _make_sc2(Q)(tx1pk, src4, dst4, w_e, zrows)

    out_full = pl.pallas_call(
        _tc_combine2,
        grid=grid,
        in_specs=[
            pl.BlockSpec((_RB, _OC), lambda i: (i, 0)),
            pl.BlockSpec((_RB, _D), lambda i: (i, 0)),
            pl.BlockSpec((_NC, _RB, _D), lambda i: (0, i, 0)),
            pl.BlockSpec((_D, _OC), lambda i: (0, 0)),
            pl.BlockSpec((_D, _OC), lambda i: (0, 0)),
        ],
        out_specs=pl.BlockSpec((_RB, 300), lambda i: (i, 0)),
        out_shape=jax.ShapeDtypeStruct((_N, 300), jnp.float32),
    )(s_acc, tx1, p2, B, C)

    return out_full


# speedup vs baseline: 1.0376x; 1.0376x over previous
import functools

import jax
import jax.numpy as jnp
import numpy as np
from jax import lax
from jax.experimental import pallas as pl
from jax.experimental.pallas import tpu as pltpu
from jax.experimental.pallas import tpu_sc as plsc

_N = 10000
_D = 128
_NC = 2
_NS = 16
_NW = _NC * _NS
_L = 16
_CH = 128
_SQ = 8
_NP = 10240
_NR = _NP // _NS

_SC_PARAMS = pltpu.CompilerParams(use_tc_tiling_on_sc=False,
                                  needs_layout_passes=False)


def _rsqrt_nr(v):
    i = plsc.bitcast(v, jnp.int32)
    y = plsc.bitcast(jnp.int32(0x5F3759DF) - lax.shift_right_logical(i, 1), jnp.float32)
    for _ in range(3):
        y = y * (1.5 - 0.5 * v * y * y)
    return y


def _scale_rows_packed(rows_pk, frows, w_b, j):
    def grp_body(g, _):
        wvec = w_b[j, pl.ds(g * _L, _L)]
        for r in range(_L):
            ws = wvec[r]
            i = g * _L + r
            for c in range(_D // (2 * _L)):
                v = rows_pk[i, pl.ds(c * _L, _L)]
                bb = plsc.bitcast(v, jnp.bfloat16)
                a, b = plsc.unpack(bb, format=plsc.PackFormat.INTERLEAVED)
                frows[i, pl.ds((2 * c) * _L, _L)] = a * ws
                frows[i, pl.ds((2 * c + 1) * _L, _L)] = b * ws
        return 0
    lax.fori_loop(0, _CH // _L, grp_body, 0)


_NB = 3


def _gather_start(h_hbm, sidx_b, rows_v, gsem, j, p):
    pltpu.async_copy(h_hbm.at[sidx_b.at[j]], rows_v.at[p], gsem.at[p])


def _gather_wait(h_hbm, sidx_b, rows_v, gsem, j, p):
    pltpu.make_async_copy(h_hbm.at[sidx_b.at[j]], rows_v.at[p],
                          gsem.at[p]).wait()


def _prop_pipelined(h_hbm, sidx_b, didx_b, w_b, rows_v, frows, acc_sh, gsem):
    for jj in range(_NB - 1):
        _gather_start(h_hbm, sidx_b, rows_v, gsem, jj, jj)

    def chunk(j, _):
        p = j % _NB
        _gather_wait(h_hbm, sidx_b, rows_v, gsem, j, p)

        @pl.when(j + _NB - 1 < _SQ)
        def _start_next_gather():
            _gather_start(h_hbm, sidx_b, rows_v, gsem, j + _NB - 1,
                          (j + _NB - 1) % _NB)

        _scale_rows_packed(rows_v.at[p], frows, w_b, j)
        pltpu.sync_copy(frows, acc_sh.at[didx_b.at[j]], add=True)
        return 0
    lax.fori_loop(0, _SQ, chunk, 0)


def _make_sc1(Q):
    mesh = plsc.VectorSubcoreMesh(core_axis_name="c", subcore_axis_name="s")

    @functools.partial(
        pl.kernel,
        out_type=(
            jax.ShapeDtypeStruct((_NW, Q, _SQ, _CH), jnp.float32),
            jax.ShapeDtypeStruct((_NC, _NP, _D), jnp.float32),
        ),
        mesh=mesh,
        compiler_params=_SC_PARAMS,
        scratch_types=[
            pltpu.VMEM_SHARED((_NP, _D), jnp.float32),
            pltpu.VMEM_SHARED((_NP,), jnp.float32),
            pltpu.VMEM_SHARED((_NP,), jnp.float32),
            pltpu.VMEM((_SQ, _CH), jnp.int32),
            pltpu.VMEM((_SQ, _CH), jnp.int32),
            pltpu.VMEM((_SQ, _CH), jnp.float32),
            pltpu.VMEM((_SQ, _CH), jnp.float32),
            pltpu.VMEM((_SQ, _CH), jnp.float32),
            pltpu.VMEM((_CH,), jnp.float32),
            pltpu.VMEM((_CH,), jnp.float32),
            pltpu.VMEM((_NR,), jnp.float32),
            pltpu.VMEM((_NB, _CH, _D // 2), jnp.float32),
            pltpu.VMEM((_CH, _D), jnp.float32),
            pltpu.SemaphoreType.DMA,
            pltpu.SemaphoreType.DMA((_NB,)),
        ],
    )
    def sc1(x_hbm, src4, dst4, ew4, zrows, zvec, w_out, p_out,
            acc_sh, deg_sh, dis_sh, sidx_b, didx_b, ew_b, w_b, ewm_b,
            dsv, ddv, disbuf, rows_v, frows, sem, gsem):
        sid = lax.axis_index("s")
        cid = lax.axis_index("c")
        wid = sid * _NC + cid
        r0 = sid * _NR

        pltpu.sync_copy(zrows, acc_sh.at[pl.ds(r0, _NR)])
        pltpu.sync_copy(zvec, deg_sh.at[pl.ds(r0, _NR)])
        plsc.subcore_barrier()

        for half in range(2):
            slc = sid + half * _NS

            def deg_sq(q, _):
                pltpu.sync_copy(src4.at[slc, q], sidx_b)
                pltpu.sync_copy(dst4.at[slc, q], didx_b)
                pltpu.sync_copy(ew4.at[slc, q], ew_b)

                def deg_chunk(j, __):
                    for g in range(_CH // _L):
                        sl16 = pl.ds(g * _L, _L)
                        s = sidx_b[j, sl16]
                        d = didx_b[j, sl16]
                        e = ew_b[j, sl16]
                        ewm_b[j, sl16] = jnp.where(s != d, e, 0.0)
                    pltpu.async_copy(ewm_b.at[j], deg_sh.at[sidx_b.at[j]],
                                     sem, add=True)
                    return 0
                lax.fori_loop(0, _SQ, deg_chunk, 0)

                def deg_drain(j, __):
                    pltpu.make_async_copy(ewm_b.at[j],
                                          deg_sh.at[sidx_b.at[j]], sem).wait()
                    return 0
                lax.fori_loop(0, _SQ, deg_drain, 0)
                return 0
            lax.fori_loop(0, Q, deg_sq, 0)
        plsc.subcore_barrier()

        pltpu.sync_copy(deg_sh.at[pl.ds(r0, _NR)], disbuf)

        def dis_body(k, _):
            acc = disbuf[pl.ds(k * _L, _L)]
            y = _rsqrt_nr(acc)
            disbuf[pl.ds(k * _L, _L)] = jnp.where(acc > 0.0, y, 0.0)
            return 0
        lax.fori_loop(0, _NR // _L, dis_body, 0)
        pltpu.sync_copy(disbuf, dis_sh.at[pl.ds(r0, _NR)])
        plsc.subcore_barrier()

        def wp_sq(q, _):
            pltpu.sync_copy(src4.at[wid, q], sidx_b)
            pltpu.sync_copy(dst4.at[wid, q], didx_b)
            pltpu.sync_copy(ew4.at[wid, q], ew_b)

            def w_chunk(j, __):
                cps = pltpu.async_copy(dis_sh.at[sidx_b.at[j]], dsv, sem)
                cpd = pltpu.async_copy(dis_sh.at[didx_b.at[j]], ddv, sem)
                cps.wait()
                cpd.wait()
                for g in range(_CH // _L):
                    sl16 = pl.ds(g * _L, _L)
                    s = sidx_b[j, sl16]
                    d = didx_b[j, sl16]
                    e = ew_b[j, sl16]
                    w_b[j, sl16] = jnp.where(s != d, (-dsv[sl16]) * e * ddv[sl16], 0.0)
                return 0
            lax.fori_loop(0, _SQ, w_chunk, 0)
            pltpu.sync_copy(w_b, w_out.at[wid, q])

            _prop_pipelined(x_hbm, sidx_b, didx_b, w_b, rows_v, frows,
                            acc_sh, gsem)
            return 0
        lax.fori_loop(0, Q, wp_sq, 0)
        plsc.subcore_barrier()

        pltpu.sync_copy(acc_sh.at[pl.ds(r0, _NR)], p_out.at[cid, pl.ds(r0, _NR)])

    return sc1


def _make_sc2(Q):
    mesh = plsc.VectorSubcoreMesh(core_axis_name="c", subcore_axis_name="s")

    @functools.partial(
        pl.kernel,
        out_type=jax.ShapeDtypeStruct((_NC, _NP, _D), jnp.float32),
        mesh=mesh,
        compiler_params=_SC_PARAMS,
        scratch_types=[
            pltpu.VMEM_SHARED((_NP, _D), jnp.float32),
            pltpu.VMEM((_SQ, _CH), jnp.int32),
            pltpu.VMEM((_SQ, _CH), jnp.int32),
            pltpu.VMEM((_SQ, _CH), jnp.float32),
            pltpu.VMEM((_NB, _CH, _D // 2), jnp.float32),
            pltpu.VMEM((_CH, _D), jnp.float32),
            pltpu.SemaphoreType.DMA((_NB,)),
        ],
    )
    def sc2(h_hbm, src4, dst4, w_hbm, zrows, p_out,
            acc_sh, sidx_b, didx_b, w_b, rows_v, frows, gsem):
        sid = lax.axis_index("s")
        cid = lax.axis_index("c")
        wid = sid * _NC + cid
        r0 = sid * _NR

        pltpu.sync_copy(zrows, acc_sh.at[pl.ds(r0, _NR)])
        plsc.subcore_barrier()

        def prop_sq(q, _):
            pltpu.sync_copy(src4.at[wid, q], sidx_b)
            pltpu.sync_copy(dst4.at[wid, q], didx_b)
            pltpu.sync_copy(w_hbm.at[wid, q], w_b)
            _prop_pipelined(h_hbm, sidx_b, didx_b, w_b, rows_v, frows,
                            acc_sh, gsem)
            return 0
        lax.fori_loop(0, Q, prop_sq, 0)
        plsc.subcore_barrier()
        pltpu.sync_copy(acc_sh.at[pl.ds(r0, _NR)], p_out.at[cid, pl.ds(r0, _NR)])

    return sc2


_RB = 1000
_OC = 384


def _tc_combine1(p1_ref, x_ref, a_ref, bias_ref, tx_ref, s_ref):
    tx_ref[...] = p1_ref[0] + p1_ref[1]
    s_ref[...] = (
        jnp.dot(x_ref[...], a_ref[...], preferred_element_type=jnp.float32)
        + bias_ref[...][0:1, :]
    )


def _tc_combine2(s_ref, tx_ref, p2_ref, b_ref, c_ref, o_ref):
    t2 = p2_ref[0] + p2_ref[1]
    o_ref[...] = (
        s_ref[...]
        + jnp.dot(tx_ref[...], b_ref[...], preferred_element_type=jnp.float32)
        + jnp.dot(t2, c_ref[...], preferred_element_type=jnp.float32)
    )


def kernel(x, edge_index, edge_weight, W1_0, b1, W2_0, W2_1, b2, W3_0, W3_1, W3_2, b3):
    E = edge_index.shape[1]
    Q = -(-E // (_NW * _SQ * _CH))
    Ep = _NW * Q * _SQ * _CH

    src = edge_index[0]
    dst = edge_index[1]
    pad = Ep - E
    src4 = jnp.pad(src, (0, pad)).reshape(_NW, Q, _SQ, _CH)
    dst4 = jnp.pad(dst, (0, pad)).reshape(_NW, Q, _SQ, _CH)
    ew4 = jnp.pad(edge_weight, (0, pad)).reshape(_NW, Q, _SQ, _CH)
    zrows = jnp.zeros((_NR, _D), jnp.float32)
    zvec = jnp.zeros((_NR,), jnp.float32)

    xpk = lax.bitcast_convert_type(
        x.astype(jnp.bfloat16).reshape(_N, _D // 2, 2), jnp.float32)
    w_e, p1 = _make_sc1(Q)(xpk, src4, dst4, ew4, zrows, zvec)

    A = jnp.pad(jnp.concatenate([W1_0, W2_0, W3_0 - W3_2], axis=1),
                ((0, 0), (0, _OC - 300)))
    B = jnp.pad(jnp.concatenate([jnp.zeros_like(W2_1), W2_1, W3_1], axis=1),
                ((0, 0), (0, _OC - 300)))
    C = jnp.pad(jnp.concatenate([jnp.zeros_like(W3_2), jnp.zeros_like(W3_2),
                                 2.0 * W3_2], axis=1), ((0, 0), (0, _OC - 300)))
    bias = jnp.broadcast_to(
        jnp.pad(jnp.concatenate([b1, b2, b3]), (0, _OC - 300)), (8, _OC))

    perm = np.zeros((_D,), np.int32)
    for c_ in range(_D // 32):
        for k_ in range(16):
            perm[32 * c_ + k_] = 32 * c_ + 2 * k_
            perm[32 * c_ + 16 + k_] = 32 * c_ + 2 * k_ + 1
    perm2 = perm[perm]
    B = B[perm, :]
    C = C[perm2, :]

    grid = (_N // _RB,)
    tx1, s_acc = pl.pallas_call(
        _tc_combine1,
        grid=grid,
        in_specs=[
            pl.BlockSpec((_NC, _RB, _D), lambda i: (0, i, 0)),
            pl.BlockSpec((_RB, _D), lambda i: (i, 0)),
            pl.BlockSpec((_D, _OC), lambda i: (0, 0)),
            pl.BlockSpec((8, _OC), lambda i: (0, 0)),
        ],
        out_specs=[
            pl.BlockSpec((_RB, _D), lambda i: (i, 0)),
            pl.BlockSpec((_RB, _OC), lambda i: (i, 0)),
        ],
        out_shape=[
            jax.ShapeDtypeStruct((_N, _D), jnp.float32),
            jax.ShapeDtypeStruct((_N, _OC), jnp.float32),
        ],
    )(p1, x, A, bias)

    tx1pk = lax.bitcast_convert_type(
        tx1.astype(jnp.bfloat16).reshape(_N, _D // 2, 2), jnp.float32)
    p2 = _make_sc2(Q)(tx1pk, src4, dst4, w_e, zrows)

    out_full = pl.pallas_call(
        _tc_combine2,
        grid=grid,
        in_specs=[
            pl.BlockSpec((_RB, _OC), lambda i: (i, 0)),
            pl.BlockSpec((_RB, _D), lambda i: (i, 0)),
            pl.BlockSpec((_NC, _RB, _D), lambda i: (0, i, 0)),
            pl.BlockSpec((_D, _OC), lambda i: (0, 0)),
            pl.BlockSpec((_D, _OC), lambda i: (0, 0)),
        ],
        out_specs=pl.BlockSpec((_RB, _OC), lambda i: (i, 0)),
        out_shape=jax.ShapeDtypeStruct((_N, _OC), jnp.float32),
    )(s_acc, tx1, p2, B, C)

    return out_full[:, :300]


# speedup vs baseline: 1.0722x; 1.0333x over previous
import functools

import jax
import jax.numpy as jnp
import numpy as np
from jax import lax
from jax.experimental import pallas as pl
from jax.experimental.pallas import tpu as pltpu
from jax.experimental.pallas import tpu_sc as plsc

_N = 10000
_D = 128
_NC = 2
_NS = 16
_NW = _NC * _NS
_L = 16
_CH = 128
_SQ = 8
_NP = 10240
_NR = _NP // _NS

_SC_PARAMS = pltpu.CompilerParams(use_tc_tiling_on_sc=False,
                                  needs_layout_passes=False)


def _rsqrt_nr(v):
    i = plsc.bitcast(v, jnp.int32)
    y = plsc.bitcast(jnp.int32(0x5F3759DF) - lax.shift_right_logical(i, 1), jnp.float32)
    for _ in range(3):
        y = y * (1.5 - 0.5 * v * y * y)
    return y


def _scale_rows_packed(rows_pk, frows, w_b, j):
    def grp_body(g, _):
        wvec = w_b[j, pl.ds(g * _L, _L)]
        for r in range(_L):
            ws = wvec[r]
            i = g * _L + r
            for c in range(_D // (2 * _L)):
                v = rows_pk[i, pl.ds(c * _L, _L)]
                bb = plsc.bitcast(v, jnp.bfloat16)
                a, b = plsc.unpack(bb, format=plsc.PackFormat.INTERLEAVED)
                frows[i, pl.ds((2 * c) * _L, _L)] = a * ws
                frows[i, pl.ds((2 * c + 1) * _L, _L)] = b * ws
        return 0
    lax.fori_loop(0, _CH // _L, grp_body, 0)


_NB = 2


def _gather_start(h_hbm, sidx_b, rows_v, gsem, j, p):
    pltpu.async_copy(h_hbm.at[sidx_b.at[j]], rows_v.at[p], gsem.at[p])


def _gather_wait(h_hbm, sidx_b, rows_v, gsem, j, p):
    pltpu.make_async_copy(h_hbm.at[sidx_b.at[j]], rows_v.at[p],
                          gsem.at[p]).wait()


def _prop_pipelined(h_hbm, sidx_b, didx_b, w_b, rows_v, frows, acc_sh, gsem):
    for jj in range(_NB - 1):
        _gather_start(h_hbm, sidx_b, rows_v, gsem, jj, jj)

    def chunk(j, _):
        p = j % _NB
        _gather_wait(h_hbm, sidx_b, rows_v, gsem, j, p)

        @pl.when(j + _NB - 1 < _SQ)
        def _start_next_gather():
            _gather_start(h_hbm, sidx_b, rows_v, gsem, j + _NB - 1,
                          (j + _NB - 1) % _NB)

        _scale_rows_packed(rows_v.at[p], frows, w_b, j)
        pltpu.sync_copy(frows, acc_sh.at[didx_b.at[j]], add=True)
        return 0
    lax.fori_loop(0, _SQ, chunk, 0)


def _make_sc1(Q):
    mesh = plsc.VectorSubcoreMesh(core_axis_name="c", subcore_axis_name="s")

    @functools.partial(
        pl.kernel,
        out_type=(
            jax.ShapeDtypeStruct((_NW, Q, _SQ, _CH), jnp.float32),
            jax.ShapeDtypeStruct((_NC, _NP, _D), jnp.float32),
        ),
        mesh=mesh,
        compiler_params=_SC_PARAMS,
        scratch_types=[
            pltpu.VMEM_SHARED((_NP, _D), jnp.float32),
            pltpu.VMEM_SHARED((_NP,), jnp.float32),
            pltpu.VMEM_SHARED((_NP,), jnp.float32),
            pltpu.VMEM((_SQ, _CH), jnp.int32),
            pltpu.VMEM((_SQ, _CH), jnp.int32),
            pltpu.VMEM((_SQ, _CH), jnp.float32),
            pltpu.VMEM((_SQ, _CH), jnp.float32),
            pltpu.VMEM((_SQ, _CH), jnp.float32),
            pltpu.VMEM((_CH,), jnp.float32),
            pltpu.VMEM((_CH,), jnp.float32),
            pltpu.VMEM((_NR,), jnp.float32),
            pltpu.VMEM((_NB, _CH, _D // 2), jnp.float32),
            pltpu.VMEM((_CH, _D), jnp.float32),
            pltpu.SemaphoreType.DMA,
            pltpu.SemaphoreType.DMA((_NB,)),
        ],
    )
    def sc1(x_hbm, src4, dst4, ew4, zrows, zvec, w_out, p_out,
            acc_sh, deg_sh, dis_sh, sidx_b, didx_b, ew_b, w_b, ewm_b,
            dsv, ddv, disbuf, rows_v, frows, sem, gsem):
        sid = lax.axis_index("s")
        cid = lax.axis_index("c")
        wid = sid * _NC + cid
        r0 = sid * _NR

        pltpu.sync_copy(zrows, acc_sh.at[pl.ds(r0, _NR)])
        pltpu.sync_copy(zvec, deg_sh.at[pl.ds(r0, _NR)])
        plsc.subcore_barrier()

        for half in range(2):
            slc = sid + half * _NS

            def deg_sq(q, _):
                pltpu.sync_copy(src4.at[slc, q], sidx_b)
                pltpu.sync_copy(dst4.at[slc, q], didx_b)
                pltpu.sync_copy(ew4.at[slc, q], ew_b)

                def deg_chunk(j, __):
                    for g in range(_CH // _L):
                        sl16 = pl.ds(g * _L, _L)
                        s = sidx_b[j, sl16]
                        d = didx_b[j, sl16]
                        e = ew_b[j, sl16]
                        ewm_b[j, sl16] = jnp.where(s != d, e, 0.0)
                    pltpu.async_copy(ewm_b.at[j], deg_sh.at[sidx_b.at[j]],
                                     sem, add=True)
                    return 0
                lax.fori_loop(0, _SQ, deg_chunk, 0)

                def deg_drain(j, __):
                    pltpu.make_async_copy(ewm_b.at[j],
                                          deg_sh.at[sidx_b.at[j]], sem).wait()
                    return 0
                lax.fori_loop(0, _SQ, deg_drain, 0)
                return 0
            lax.fori_loop(0, Q, deg_sq, 0)
        plsc.subcore_barrier()

        pltpu.sync_copy(deg_sh.at[pl.ds(r0, _NR)], disbuf)

        def dis_body(k, _):
            acc = disbuf[pl.ds(k * _L, _L)]
            y = _rsqrt_nr(acc)
            disbuf[pl.ds(k * _L, _L)] = jnp.where(acc > 0.0, y, 0.0)
            return 0
        lax.fori_loop(0, _NR // _L, dis_body, 0)
        pltpu.sync_copy(disbuf, dis_sh.at[pl.ds(r0, _NR)])
        plsc.subcore_barrier()

        def wp_sq(q, _):
            pltpu.sync_copy(src4.at[wid, q], sidx_b)
            pltpu.sync_copy(dst4.at[wid, q], didx_b)
            pltpu.sync_copy(ew4.at[wid, q], ew_b)

            def w_chunk(j, __):
                cps = pltpu.async_copy(dis_sh.at[sidx_b.at[j]], dsv, sem)
                cpd = pltpu.async_copy(dis_sh.at[didx_b.at[j]], ddv, sem)
                cps.wait()
                cpd.wait()
                for g in range(_CH // _L):
                    sl16 = pl.ds(g * _L, _L)
                    s = sidx_b[j, sl16]
                    d = didx_b[j, sl16]
                    e = ew_b[j, sl16]
                    w_b[j, sl16] = jnp.where(s != d, (-dsv[sl16]) * e * ddv[sl16], 0.0)
                return 0
            lax.fori_loop(0, _SQ, w_chunk, 0)
            pltpu.sync_copy(w_b, w_out.at[wid, q])

            _prop_pipelined(x_hbm, sidx_b, didx_b, w_b, rows_v, frows,
                            acc_sh, gsem)
            return 0
        lax.fori_loop(0, Q, wp_sq, 0)
        plsc.subcore_barrier()

        pltpu.sync_copy(acc_sh.at[pl.ds(r0, _NR)], p_out.at[cid, pl.ds(r0, _NR)])

    return sc1


def _make_sc2(Q):
    mesh = plsc.VectorSubcoreMesh(core_axis_name="c", subcore_axis_name="s")

    @functools.partial(
        pl.kernel,
        out_type=jax.ShapeDtypeStruct((_NC, _NP, _D), jnp.float32),
        mesh=mesh,
        compiler_params=_SC_PARAMS,
        scratch_types=[
            pltpu.VMEM_SHARED((_NP, _D), jnp.float32),
            pltpu.VMEM((_SQ, _CH), jnp.int32),
            pltpu.VMEM((_SQ, _CH), jnp.int32),
            pltpu.VMEM((_SQ, _CH), jnp.float32),
            pltpu.VMEM((_NB, _CH, _D // 2), jnp.float32),
            pltpu.VMEM((_CH, _D), jnp.float32),
            pltpu.SemaphoreType.DMA((_NB,)),
        ],
    )
    def sc2(h_hbm, src4, dst4, w_hbm, zrows, p_out,
            acc_sh, sidx_b, didx_b, w_b, rows_v, frows, gsem):
        sid = lax.axis_index("s")
        cid = lax.axis_index("c")
        wid = sid * _NC + cid
        r0 = sid * _NR

        pltpu.sync_copy(zrows, acc_sh.at[pl.ds(r0, _NR)])
        plsc.subcore_barrier()

        def prop_sq(q, _):
            pltpu.sync_copy(src4.at[wid, q], sidx_b)
            pltpu.sync_copy(dst4.at[wid, q], didx_b)
            pltpu.sync_copy(w_hbm.at[wid, q], w_b)
            _prop_pipelined(h_hbm, sidx_b, didx_b, w_b, rows_v, frows,
                            acc_sh, gsem)
            return 0
        lax.fori_loop(0, Q, prop_sq, 0)
        plsc.subcore_barrier()
        pltpu.sync_copy(acc_sh.at[pl.ds(r0, _NR)], p_out.at[cid, pl.ds(r0, _NR)])

    return sc2


_RB = 1000
_OC = 384


def _tc_combine1(p1_ref, x_ref, a_ref, bias_ref, tx_ref, s_ref):
    tx_ref[...] = p1_ref[0] + p1_ref[1]
    s_ref[...] = (
        jnp.dot(x_ref[...], a_ref[...], preferred_element_type=jnp.float32)
        + bias_ref[...][0:1, :]
    )


def _tc_combine2(s_ref, tx_ref, p2_ref, b_ref, c_ref, o_ref):
    t2 = p2_ref[0] + p2_ref[1]
    o_ref[...] = (
        s_ref[...]
        + jnp.dot(tx_ref[...], b_ref[...], preferred_element_type=jnp.float32)
        + jnp.dot(t2, c_ref[...], preferred_element_type=jnp.float32)
    )


def kernel(x, edge_index, edge_weight, W1_0, b1, W2_0, W2_1, b2, W3_0, W3_1, W3_2, b3):
    E = edge_index.shape[1]
    Q = -(-E // (_NW * _SQ * _CH))
    Ep = _NW * Q * _SQ * _CH

    src = edge_index[0]
    dst = edge_index[1]
    pad = Ep - E
    src4 = jnp.pad(src, (0, pad)).reshape(_NW, Q, _SQ, _CH)
    dst4 = jnp.pad(dst, (0, pad)).reshape(_NW, Q, _SQ, _CH)
    ew4 = jnp.pad(edge_weight, (0, pad)).reshape(_NW, Q, _SQ, _CH)
    zrows = jnp.zeros((_NR, _D), jnp.float32)
    zvec = jnp.zeros((_NR,), jnp.float32)

    xpk = lax.bitcast_convert_type(
        x.astype(jnp.bfloat16).reshape(_N, _D // 2, 2), jnp.float32)
    w_e, p1 = _make_sc1(Q)(xpk, src4, dst4, ew4, zrows, zvec)

    A = jnp.pad(jnp.concatenate([W1_0, W2_0, W3_0 - W3_2], axis=1),
                ((0, 0), (0, _OC - 300)))
    B = jnp.pad(jnp.concatenate([jnp.zeros_like(W2_1), W2_1, W3_1], axis=1),
                ((0, 0), (0, _OC - 300)))
    C = jnp.pad(jnp.concatenate([jnp.zeros_like(W3_2), jnp.zeros_like(W3_2),
                                 2.0 * W3_2], axis=1), ((0, 0), (0, _OC - 300)))
    bias = jnp.broadcast_to(
        jnp.pad(jnp.concatenate([b1, b2, b3]), (0, _OC - 300)), (8, _OC))

    perm = np.zeros((_D,), np.int32)
    for c_ in range(_D // 32):
        for k_ in range(16):
            perm[32 * c_ + k_] = 32 * c_ + 2 * k_
            perm[32 * c_ + 16 + k_] = 32 * c_ + 2 * k_ + 1
    perm2 = perm[perm]
    B = B[perm, :]
    C = C[perm2, :]

    grid = (_N // _RB,)
    tx1, s_acc = pl.pallas_call(
        _tc_combine1,
        grid=grid,
        in_specs=[
            pl.BlockSpec((_NC, _RB, _D), lambda i: (0, i, 0)),
            pl.BlockSpec((_RB, _D), lambda i: (i, 0)),
            pl.BlockSpec((_D, _OC), lambda i: (0, 0)),
            pl.BlockSpec((8, _OC), lambda i: (0, 0)),
        ],
        out_specs=[
            pl.BlockSpec((_RB, _D), lambda i: (i, 0)),
            pl.BlockSpec((_RB, _OC), lambda i: (i, 0)),
        ],
        out_shape=[
            jax.ShapeDtypeStruct((_N, _D), jnp.float32),
            jax.ShapeDtypeStruct((_N, _OC), jnp.float32),
        ],
    )(p1, x, A, bias)

    tx1pk = lax.bitcast_convert_type(
        tx1.astype(jnp.bfloat16).reshape(_N, _D // 2, 2), jnp.float32)
    p2 = _make_sc2(Q)(tx1pk, src4, dst4, w_e, zrows)

    out_full = pl.pallas_call(
        _tc_combine2,
        grid=grid,
        in_specs=[
            pl.BlockSpec((_RB, _OC), lambda i: (i, 0)),
            pl.BlockSpec((_RB, _D), lambda i: (i, 0)),
            pl.BlockSpec((_NC, _RB, _D), lambda i: (0, i, 0)),
            pl.BlockSpec((_D, _OC), lambda i: (0, 0)),
            pl.BlockSpec((_D, _OC), lambda i: (0, 0)),
        ],
        out_specs=pl.BlockSpec((_RB, _OC), lambda i: (i, 0)),
        out_shape=jax.ShapeDtypeStruct((_N, _OC), jnp.float32),
    )(s_acc, tx1, p2, B, C)

    return out_full[:, :300]


# speedup vs baseline: 1.1377x; 1.0611x over previous
import functools

import jax
import jax.numpy as jnp
import numpy as np
from jax import lax
from jax.experimental import pallas as pl
from jax.experimental.pallas import tpu as pltpu
from jax.experimental.pallas import tpu_sc as plsc

_N = 10000
_D = 128
_NC = 2
_NS = 16
_NW = _NC * _NS
_L = 16
_CH = 128
_SQ = 16
_NP = 10240
_NR = _NP // _NS

_SC_PARAMS = pltpu.CompilerParams(use_tc_tiling_on_sc=False,
                                  needs_layout_passes=False)


def _rsqrt_nr(v):
    i = plsc.bitcast(v, jnp.int32)
    y = plsc.bitcast(jnp.int32(0x5F3759DF) - lax.shift_right_logical(i, 1), jnp.float32)
    for _ in range(3):
        y = y * (1.5 - 0.5 * v * y * y)
    return y


def _scale_rows_packed(rows_pk, frows, w_b, j):
    def grp_body(g, _):
        wvec = w_b[j, pl.ds(g * _L, _L)]
        for r in range(_L):
            ws = wvec[r]
            i = g * _L + r
            for c in range(_D // (2 * _L)):
                v = rows_pk[i, pl.ds(c * _L, _L)]
                bb = plsc.bitcast(v, jnp.bfloat16)
                a, b = plsc.unpack(bb, format=plsc.PackFormat.INTERLEAVED)
                frows[i, pl.ds((2 * c) * _L, _L)] = a * ws
                frows[i, pl.ds((2 * c + 1) * _L, _L)] = b * ws
        return 0
    lax.fori_loop(0, _CH // _L, grp_body, 0)


_NB = 2


def _gather_start(h_hbm, sidx_b, rows_v, gsem, j, p):
    pltpu.async_copy(h_hbm.at[sidx_b.at[j]], rows_v.at[p], gsem.at[p])


def _gather_wait(h_hbm, sidx_b, rows_v, gsem, j, p):
    pltpu.make_async_copy(h_hbm.at[sidx_b.at[j]], rows_v.at[p],
                          gsem.at[p]).wait()


def _prop_pipelined(h_hbm, sidx_b, didx_b, w_b, rows_v, frows, acc_sh, gsem):
    for jj in range(_NB - 1):
        _gather_start(h_hbm, sidx_b, rows_v, gsem, jj, jj)

    def chunk(j, _):
        p = j % _NB
        _gather_wait(h_hbm, sidx_b, rows_v, gsem, j, p)

        @pl.when(j + _NB - 1 < _SQ)
        def _start_next_gather():
            _gather_start(h_hbm, sidx_b, rows_v, gsem, j + _NB - 1,
                          (j + _NB - 1) % _NB)

        _scale_rows_packed(rows_v.at[p], frows, w_b, j)
        pltpu.sync_copy(frows, acc_sh.at[didx_b.at[j]], add=True)
        return 0
    lax.fori_loop(0, _SQ, chunk, 0)


def _make_sc1(Q):
    mesh = plsc.VectorSubcoreMesh(core_axis_name="c", subcore_axis_name="s")

    @functools.partial(
        pl.kernel,
        out_type=(
            jax.ShapeDtypeStruct((_NW, Q, _SQ, _CH), jnp.float32),
            jax.ShapeDtypeStruct((_NC, _NP, _D), jnp.float32),
        ),
        mesh=mesh,
        compiler_params=_SC_PARAMS,
        scratch_types=[
            pltpu.VMEM_SHARED((_NP, _D), jnp.float32),
            pltpu.VMEM_SHARED((_NP,), jnp.float32),
            pltpu.VMEM_SHARED((_NP,), jnp.float32),
            pltpu.VMEM((_SQ, _CH), jnp.int32),
            pltpu.VMEM((_SQ, _CH), jnp.int32),
            pltpu.VMEM((_SQ, _CH), jnp.float32),
            pltpu.VMEM((_SQ, _CH), jnp.float32),
            pltpu.VMEM((_SQ, _CH), jnp.float32),
            pltpu.VMEM((_CH,), jnp.float32),
            pltpu.VMEM((_CH,), jnp.float32),
            pltpu.VMEM((_NR,), jnp.float32),
            pltpu.VMEM((_NB, _CH, _D // 2), jnp.float32),
            pltpu.VMEM((_CH, _D), jnp.float32),
            pltpu.SemaphoreType.DMA,
            pltpu.SemaphoreType.DMA((_NB,)),
        ],
    )
    def sc1(x_hbm, src4, dst4, ew4, zrows, zvec, w_out, p_out,
            acc_sh, deg_sh, dis_sh, sidx_b, didx_b, ew_b, w_b, ewm_b,
            dsv, ddv, disbuf, rows_v, frows, sem, gsem):
        sid = lax.axis_index("s")
        cid = lax.axis_index("c")
        wid = sid * _NC + cid
        r0 = sid * _NR

        pltpu.sync_copy(zrows, acc_sh.at[pl.ds(r0, _NR)])
        pltpu.sync_copy(zvec, deg_sh.at[pl.ds(r0, _NR)])
        plsc.subcore_barrier()

        for half in range(2):
            slc = sid + half * _NS

            def deg_sq(q, _):
                pltpu.sync_copy(src4.at[slc, q], sidx_b)
                pltpu.sync_copy(dst4.at[slc, q], didx_b)
                pltpu.sync_copy(ew4.at[slc, q], ew_b)

                def deg_chunk(j, __):
                    for g in range(_CH // _L):
                        sl16 = pl.ds(g * _L, _L)
                        s = sidx_b[j, sl16]
                        d = didx_b[j, sl16]
                        e = ew_b[j, sl16]
                        ewm_b[j, sl16] = jnp.where(s != d, e, 0.0)
                    pltpu.async_copy(ewm_b.at[j], deg_sh.at[sidx_b.at[j]],
                                     sem, add=True)
                    return 0
                lax.fori_loop(0, _SQ, deg_chunk, 0)

                def deg_drain(j, __):
                    pltpu.make_async_copy(ewm_b.at[j],
                                          deg_sh.at[sidx_b.at[j]], sem).wait()
                    return 0
                lax.fori_loop(0, _SQ, deg_drain, 0)
                return 0
            lax.fori_loop(0, Q, deg_sq, 0)
        plsc.subcore_barrier()

        pltpu.sync_copy(deg_sh.at[pl.ds(r0, _NR)], disbuf)

        def dis_body(k, _):
            acc = disbuf[pl.ds(k * _L, _L)]
            y = _rsqrt_nr(acc)
            disbuf[pl.ds(k * _L, _L)] = jnp.where(acc > 0.0, y, 0.0)
            return 0
        lax.fori_loop(0, _NR // _L, dis_body, 0)
        pltpu.sync_copy(disbuf, dis_sh.at[pl.ds(r0, _NR)])
        plsc.subcore_barrier()

        def wp_sq(q, _):
            pltpu.sync_copy(src4.at[wid, q], sidx_b)
            pltpu.sync_copy(dst4.at[wid, q], didx_b)
            pltpu.sync_copy(ew4.at[wid, q], ew_b)

            def w_chunk(j, __):
                cps = pltpu.async_copy(dis_sh.at[sidx_b.at[j]], dsv, sem)
                cpd = pltpu.async_copy(dis_sh.at[didx_b.at[j]], ddv, sem)
                cps.wait()
                cpd.wait()
                for g in range(_CH // _L):
                    sl16 = pl.ds(g * _L, _L)
                    s = sidx_b[j, sl16]
                    d = didx_b[j, sl16]
                    e = ew_b[j, sl16]
                    w_b[j, sl16] = jnp.where(s != d, (-dsv[sl16]) * e * ddv[sl16], 0.0)
                return 0
            lax.fori_loop(0, _SQ, w_chunk, 0)
            pltpu.sync_copy(w_b, w_out.at[wid, q])

            _prop_pipelined(x_hbm, sidx_b, didx_b, w_b, rows_v, frows,
                            acc_sh, gsem)
            return 0
        lax.fori_loop(0, Q, wp_sq, 0)
        plsc.subcore_barrier()

        pltpu.sync_copy(acc_sh.at[pl.ds(r0, _NR)], p_out.at[cid, pl.ds(r0, _NR)])

    return sc1


def _make_sc2(Q):
    mesh = plsc.VectorSubcoreMesh(core_axis_name="c", subcore_axis_name="s")

    @functools.partial(
        pl.kernel,
        out_type=jax.ShapeDtypeStruct((_NC, _NP, _D), jnp.float32),
        mesh=mesh,
        compiler_params=_SC_PARAMS,
        scratch_types=[
            pltpu.VMEM_SHARED((_NP, _D), jnp.float32),
            pltpu.VMEM((_SQ, _CH), jnp.int32),
            pltpu.VMEM((_SQ, _CH), jnp.int32),
            pltpu.VMEM((_SQ, _CH), jnp.float32),
            pltpu.VMEM((_NB, _CH, _D // 2), jnp.float32),
            pltpu.VMEM((_CH, _D), jnp.float32),
            pltpu.SemaphoreType.DMA((_NB,)),
        ],
    )
    def sc2(h_hbm, src4, dst4, w_hbm, zrows, p_out,
            acc_sh, sidx_b, didx_b, w_b, rows_v, frows, gsem):
        sid = lax.axis_index("s")
        cid = lax.axis_index("c")
        wid = sid * _NC + cid
        r0 = sid * _NR

        pltpu.sync_copy(zrows, acc_sh.at[pl.ds(r0, _NR)])
        plsc.subcore_barrier()

        def prop_sq(q, _):
            pltpu.sync_copy(src4.at[wid, q], sidx_b)
            pltpu.sync_copy(dst4.at[wid, q], didx_b)
            pltpu.sync_copy(w_hbm.at[wid, q], w_b)
            _prop_pipelined(h_hbm, sidx_b, didx_b, w_b, rows_v, frows,
                            acc_sh, gsem)
            return 0
        lax.fori_loop(0, Q, prop_sq, 0)
        plsc.subcore_barrier()
        pltpu.sync_copy(acc_sh.at[pl.ds(r0, _NR)], p_out.at[cid, pl.ds(r0, _NR)])

    return sc2


_RB = 1000
_OC = 384


def _tc_combine1(p1_ref, x_ref, a_ref, bias_ref, tx_ref, s_ref):
    tx_ref[...] = p1_ref[0] + p1_ref[1]
    s_ref[...] = (
        jnp.dot(x_ref[...], a_ref[...], preferred_element_type=jnp.float32)
        + bias_ref[...][0:1, :]
    )


def _tc_combine2(s_ref, tx_ref, p2_ref, b_ref, c_ref, o_ref):
    t2 = p2_ref[0] + p2_ref[1]
    o_ref[...] = (
        s_ref[...]
        + jnp.dot(tx_ref[...], b_ref[...], preferred_element_type=jnp.float32)
        + jnp.dot(t2, c_ref[...], preferred_element_type=jnp.float32)
    )


def kernel(x, edge_index, edge_weight, W1_0, b1, W2_0, W2_1, b2, W3_0, W3_1, W3_2, b3):
    E = edge_index.shape[1]
    Q = -(-E // (_NW * _SQ * _CH))
    Ep = _NW * Q * _SQ * _CH

    src = edge_index[0]
    dst = edge_index[1]
    pad = Ep - E
    src4 = jnp.pad(src, (0, pad)).reshape(_NW, Q, _SQ, _CH)
    dst4 = jnp.pad(dst, (0, pad)).reshape(_NW, Q, _SQ, _CH)
    ew4 = jnp.pad(edge_weight, (0, pad)).reshape(_NW, Q, _SQ, _CH)
    zrows = jnp.zeros((_NR, _D), jnp.float32)
    zvec = jnp.zeros((_NR,), jnp.float32)

    xpk = lax.bitcast_convert_type(
        x.astype(jnp.bfloat16).reshape(_N, _D // 2, 2), jnp.float32)
    w_e, p1 = _make_sc1(Q)(xpk, src4, dst4, ew4, zrows, zvec)

    A = jnp.pad(jnp.concatenate([W1_0, W2_0, W3_0 - W3_2], axis=1),
                ((0, 0), (0, _OC - 300)))
    B = jnp.pad(jnp.concatenate([jnp.zeros_like(W2_1), W2_1, W3_1], axis=1),
                ((0, 0), (0, _OC - 300)))
    C = jnp.pad(jnp.concatenate([jnp.zeros_like(W3_2), jnp.zeros_like(W3_2),
                                 2.0 * W3_2], axis=1), ((0, 0), (0, _OC - 300)))
    bias = jnp.broadcast_to(
        jnp.pad(jnp.concatenate([b1, b2, b3]), (0, _OC - 300)), (8, _OC))

    perm = np.zeros((_D,), np.int32)
    for c_ in range(_D // 32):
        for k_ in range(16):
            perm[32 * c_ + k_] = 32 * c_ + 2 * k_
            perm[32 * c_ + 16 + k_] = 32 * c_ + 2 * k_ + 1
    perm2 = perm[perm]
    B = B[perm, :]
    C = C[perm2, :]

    grid = (_N // _RB,)
    tx1, s_acc = pl.pallas_call(
        _tc_combine1,
        grid=grid,
        in_specs=[
            pl.BlockSpec((_NC, _RB, _D), lambda i: (0, i, 0)),
            pl.BlockSpec((_RB, _D), lambda i: (i, 0)),
            pl.BlockSpec((_D, _OC), lambda i: (0, 0)),
            pl.BlockSpec((8, _OC), lambda i: (0, 0)),
        ],
        out_specs=[
            pl.BlockSpec((_RB, _D), lambda i: (i, 0)),
            pl.BlockSpec((_RB, _OC), lambda i: (i, 0)),
        ],
        out_shape=[
            jax.ShapeDtypeStruct((_N, _D), jnp.float32),
            jax.ShapeDtypeStruct((_N, _OC), jnp.float32),
        ],
    )(p1, x, A, bias)

    tx1pk = lax.bitcast_convert_type(
        tx1.astype(jnp.bfloat16).reshape(_N, _D // 2, 2), jnp.float32)
    p2 = _make_sc2(Q)(tx1pk, src4, dst4, w_e, zrows)

    out_full = pl.pallas_call(
        _tc_combine2,
        grid=grid,
        in_specs=[
            pl.BlockSpec((_RB, _OC), lambda i: (i, 0)),
            pl.BlockSpec((_RB, _D), lambda i: (i, 0)),
            pl.BlockSpec((_NC, _RB, _D), lambda i: (0, i, 0)),
            pl.BlockSpec((_D, _OC), lambda i: (0, 0)),
            pl.BlockSpec((_D, _OC), lambda i: (0, 0)),
        ],
        out_specs=pl.BlockSpec((_RB, _OC), lambda i: (i, 0)),
        out_shape=jax.ShapeDtypeStruct((_N, _OC), jnp.float32),
    )(s_acc, tx1, p2, B, C)

    return out_full[:, :300]


# speedup vs baseline: 1.1463x; 1.0075x over previous
import functools

import jax
import jax.numpy as jnp
import numpy as np
from jax import lax
from jax.experimental import pallas as pl
from jax.experimental.pallas import tpu as pltpu
from jax.experimental.pallas import tpu_sc as plsc

_N = 10000
_D = 128
_NC = 2
_NS = 16
_NW = _NC * _NS
_L = 16
_CH = 128
_SQ = 20
_NP = 10240
_NR = _NP // _NS

_SC_PARAMS = pltpu.CompilerParams(use_tc_tiling_on_sc=False,
                                  needs_layout_passes=False)


def _rsqrt_nr(v):
    i = plsc.bitcast(v, jnp.int32)
    y = plsc.bitcast(jnp.int32(0x5F3759DF) - lax.shift_right_logical(i, 1), jnp.float32)
    for _ in range(3):
        y = y * (1.5 - 0.5 * v * y * y)
    return y


def _scale_rows_packed(rows_pk, frows, w_b, j):
    def grp_body(g, _):
        wvec = w_b[j, pl.ds(g * _L, _L)]
        for r in range(_L):
            ws = wvec[r]
            i = g * _L + r
            for c in range(_D // (2 * _L)):
                v = rows_pk[i, pl.ds(c * _L, _L)]
                bb = plsc.bitcast(v, jnp.bfloat16)
                a, b = plsc.unpack(bb, format=plsc.PackFormat.INTERLEAVED)
                frows[i, pl.ds((2 * c) * _L, _L)] = a * ws
                frows[i, pl.ds((2 * c + 1) * _L, _L)] = b * ws
        return 0
    lax.fori_loop(0, _CH // _L, grp_body, 0)


_NB = 2


def _gather_start(h_hbm, sidx_b, rows_v, gsem, j, p):
    pltpu.async_copy(h_hbm.at[sidx_b.at[j]], rows_v.at[p], gsem.at[p])


def _gather_wait(h_hbm, sidx_b, rows_v, gsem, j, p):
    pltpu.make_async_copy(h_hbm.at[sidx_b.at[j]], rows_v.at[p],
                          gsem.at[p]).wait()


def _prop_pipelined(h_hbm, sidx_b, didx_b, w_b, rows_v, frows, acc_sh, gsem):
    for jj in range(_NB - 1):
        _gather_start(h_hbm, sidx_b, rows_v, gsem, jj, jj)

    def chunk(j, _):
        p = j % _NB
        _gather_wait(h_hbm, sidx_b, rows_v, gsem, j, p)

        @pl.when(j + _NB - 1 < _SQ)
        def _start_next_gather():
            _gather_start(h_hbm, sidx_b, rows_v, gsem, j + _NB - 1,
                          (j + _NB - 1) % _NB)

        _scale_rows_packed(rows_v.at[p], frows, w_b, j)
        pltpu.sync_copy(frows, acc_sh.at[didx_b.at[j]], add=True)
        return 0
    lax.fori_loop(0, _SQ, chunk, 0)


def _make_sc1(Q):
    mesh = plsc.VectorSubcoreMesh(core_axis_name="c", subcore_axis_name="s")

    @functools.partial(
        pl.kernel,
        out_type=(
            jax.ShapeDtypeStruct((_NW, Q, _SQ, _CH), jnp.float32),
            jax.ShapeDtypeStruct((_NC, _NP, _D), jnp.float32),
        ),
        mesh=mesh,
        compiler_params=_SC_PARAMS,
        scratch_types=[
            pltpu.VMEM_SHARED((_NP, _D), jnp.float32),
            pltpu.VMEM_SHARED((_NP,), jnp.float32),
            pltpu.VMEM_SHARED((_NP,), jnp.float32),
            pltpu.VMEM((_SQ, _CH), jnp.int32),
            pltpu.VMEM((_SQ, _CH), jnp.int32),
            pltpu.VMEM((_SQ, _CH), jnp.float32),
            pltpu.VMEM((_SQ, _CH), jnp.float32),
            pltpu.VMEM((_SQ, _CH), jnp.float32),
            pltpu.VMEM((_CH,), jnp.float32),
            pltpu.VMEM((_CH,), jnp.float32),
            pltpu.VMEM((_NR,), jnp.float32),
            pltpu.VMEM((_NB, _CH, _D // 2), jnp.float32),
            pltpu.VMEM((_CH, _D), jnp.float32),
            pltpu.SemaphoreType.DMA,
            pltpu.SemaphoreType.DMA((_NB,)),
        ],
    )
    def sc1(x_hbm, src4, dst4, ew4, zrows, zvec, w_out, p_out,
            acc_sh, deg_sh, dis_sh, sidx_b, didx_b, ew_b, w_b, ewm_b,
            dsv, ddv, disbuf, rows_v, frows, sem, gsem):
        sid = lax.axis_index("s")
        cid = lax.axis_index("c")
        wid = sid * _NC + cid
        r0 = sid * _NR

        pltpu.sync_copy(zrows, acc_sh.at[pl.ds(r0, _NR)])
        pltpu.sync_copy(zvec, deg_sh.at[pl.ds(r0, _NR)])
        plsc.subcore_barrier()

        for half in range(2):
            slc = sid + half * _NS

            def deg_sq(q, _):
                pltpu.sync_copy(src4.at[slc, q], sidx_b)
                pltpu.sync_copy(dst4.at[slc, q], didx_b)
                pltpu.sync_copy(ew4.at[slc, q], ew_b)

                def deg_chunk(j, __):
                    for g in range(_CH // _L):
                        sl16 = pl.ds(g * _L, _L)
                        s = sidx_b[j, sl16]
                        d = didx_b[j, sl16]
                        e = ew_b[j, sl16]
                        ewm_b[j, sl16] = jnp.where(s != d, e, 0.0)
                    pltpu.async_copy(ewm_b.at[j], deg_sh.at[sidx_b.at[j]],
                                     sem, add=True)
                    return 0
                lax.fori_loop(0, _SQ, deg_chunk, 0)

                def deg_drain(j, __):
                    pltpu.make_async_copy(ewm_b.at[j],
                                          deg_sh.at[sidx_b.at[j]], sem).wait()
                    return 0
                lax.fori_loop(0, _SQ, deg_drain, 0)
                return 0
            lax.fori_loop(0, Q, deg_sq, 0)
        plsc.subcore_barrier()

        pltpu.sync_copy(deg_sh.at[pl.ds(r0, _NR)], disbuf)

        def dis_body(k, _):
            acc = disbuf[pl.ds(k * _L, _L)]
            y = _rsqrt_nr(acc)
            disbuf[pl.ds(k * _L, _L)] = jnp.where(acc > 0.0, y, 0.0)
            return 0
        lax.fori_loop(0, _NR // _L, dis_body, 0)
        pltpu.sync_copy(disbuf, dis_sh.at[pl.ds(r0, _NR)])
        plsc.subcore_barrier()

        def wp_sq(q, _):
            pltpu.sync_copy(src4.at[wid, q], sidx_b)
            pltpu.sync_copy(dst4.at[wid, q], didx_b)
            pltpu.sync_copy(ew4.at[wid, q], ew_b)

            def w_chunk(j, __):
                cps = pltpu.async_copy(dis_sh.at[sidx_b.at[j]], dsv, sem)
                cpd = pltpu.async_copy(dis_sh.at[didx_b.at[j]], ddv, sem)
                cps.wait()
                cpd.wait()
                for g in range(_CH // _L):
                    sl16 = pl.ds(g * _L, _L)
                    s = sidx_b[j, sl16]
                    d = didx_b[j, sl16]
                    e = ew_b[j, sl16]
                    w_b[j, sl16] = jnp.where(s != d, (-dsv[sl16]) * e * ddv[sl16], 0.0)
                return 0
            lax.fori_loop(0, _SQ, w_chunk, 0)
            pltpu.sync_copy(w_b, w_out.at[wid, q])

            _prop_pipelined(x_hbm, sidx_b, didx_b, w_b, rows_v, frows,
                            acc_sh, gsem)
            return 0
        lax.fori_loop(0, Q, wp_sq, 0)
        plsc.subcore_barrier()

        pltpu.sync_copy(acc_sh.at[pl.ds(r0, _NR)], p_out.at[cid, pl.ds(r0, _NR)])

    return sc1


def _make_sc2(Q):
    mesh = plsc.VectorSubcoreMesh(core_axis_name="c", subcore_axis_name="s")

    @functools.partial(
        pl.kernel,
        out_type=jax.ShapeDtypeStruct((_NC, _NP, _D), jnp.float32),
        mesh=mesh,
        compiler_params=_SC_PARAMS,
        scratch_types=[
            pltpu.VMEM_SHARED((_NP, _D), jnp.float32),
            pltpu.VMEM((_SQ, _CH), jnp.int32),
            pltpu.VMEM((_SQ, _CH), jnp.int32),
            pltpu.VMEM((_SQ, _CH), jnp.float32),
            pltpu.VMEM((_NB, _CH, _D // 2), jnp.float32),
            pltpu.VMEM((_CH, _D), jnp.float32),
            pltpu.SemaphoreType.DMA((_NB,)),
        ],
    )
    def sc2(h_hbm, src4, dst4, w_hbm, zrows, p_out,
            acc_sh, sidx_b, didx_b, w_b, rows_v, frows, gsem):
        sid = lax.axis_index("s")
        cid = lax.axis_index("c")
        wid = sid * _NC + cid
        r0 = sid * _NR

        pltpu.sync_copy(zrows, acc_sh.at[pl.ds(r0, _NR)])
        plsc.subcore_barrier()

        def prop_sq(q, _):
            pltpu.sync_copy(src4.at[wid, q], sidx_b)
            pltpu.sync_copy(dst4.at[wid, q], didx_b)
            pltpu.sync_copy(w_hbm.at[wid, q], w_b)
            _prop_pipelined(h_hbm, sidx_b, didx_b, w_b, rows_v, frows,
                            acc_sh, gsem)
            return 0
        lax.fori_loop(0, Q, prop_sq, 0)
        plsc.subcore_barrier()
        pltpu.sync_copy(acc_sh.at[pl.ds(r0, _NR)], p_out.at[cid, pl.ds(r0, _NR)])

    return sc2


_RB = 1000
_OC = 384


def _tc_combine1(p1_ref, x_ref, a_ref, bias_ref, tx_ref, s_ref):
    tx_ref[...] = p1_ref[0] + p1_ref[1]
    s_ref[...] = (
        jnp.dot(x_ref[...], a_ref[...], preferred_element_type=jnp.float32)
        + bias_ref[...][0:1, :]
    )


def _tc_combine2(s_ref, tx_ref, p2_ref, b_ref, c_ref, o_ref):
    t2 = p2_ref[0] + p2_ref[1]
    o_ref[...] = (
        s_ref[...]
        + jnp.dot(tx_ref[...], b_ref[...], preferred_element_type=jnp.float32)
        + jnp.dot(t2, c_ref[...], preferred_element_type=jnp.float32)
    )


def kernel(x, edge_index, edge_weight, W1_0, b1, W2_0, W2_1, b2, W3_0, W3_1, W3_2, b3):
    E = edge_index.shape[1]
    Q = -(-E // (_NW * _SQ * _CH))
    Ep = _NW * Q * _SQ * _CH

    src = edge_index[0]
    dst = edge_index[1]
    pad = Ep - E
    src4 = jnp.pad(src, (0, pad)).reshape(_NW, Q, _SQ, _CH)
    dst4 = jnp.pad(dst, (0, pad)).reshape(_NW, Q, _SQ, _CH)
    ew4 = jnp.pad(edge_weight, (0, pad)).reshape(_NW, Q, _SQ, _CH)
    zrows = jnp.zeros((_NR, _D), jnp.float32)
    zvec = jnp.zeros((_NR,), jnp.float32)

    xpk = lax.bitcast_convert_type(
        x.astype(jnp.bfloat16).reshape(_N, _D // 2, 2), jnp.float32)
    w_e, p1 = _make_sc1(Q)(xpk, src4, dst4, ew4, zrows, zvec)

    A = jnp.pad(jnp.concatenate([W1_0, W2_0, W3_0 - W3_2], axis=1),
                ((0, 0), (0, _OC - 300)))
    B = jnp.pad(jnp.concatenate([jnp.zeros_like(W2_1), W2_1, W3_1], axis=1),
                ((0, 0), (0, _OC - 300)))
    C = jnp.pad(jnp.concatenate([jnp.zeros_like(W3_2), jnp.zeros_like(W3_2),
                                 2.0 * W3_2], axis=1), ((0, 0), (0, _OC - 300)))
    bias = jnp.broadcast_to(
        jnp.pad(jnp.concatenate([b1, b2, b3]), (0, _OC - 300)), (8, _OC))

    perm = np.zeros((_D,), np.int32)
    for c_ in range(_D // 32):
        for k_ in range(16):
            perm[32 * c_ + k_] = 32 * c_ + 2 * k_
            perm[32 * c_ + 16 + k_] = 32 * c_ + 2 * k_ + 1
    perm2 = perm[perm]
    B = B[perm, :]
    C = C[perm2, :]

    grid = (_N // _RB,)
    tx1, s_acc = pl.pallas_call(
        _tc_combine1,
        grid=grid,
        in_specs=[
            pl.BlockSpec((_NC, _RB, _D), lambda i: (0, i, 0)),
            pl.BlockSpec((_RB, _D), lambda i: (i, 0)),
            pl.BlockSpec((_D, _OC), lambda i: (0, 0)),
            pl.BlockSpec((8, _OC), lambda i: (0, 0)),
        ],
        out_specs=[
            pl.BlockSpec((_RB, _D), lambda i: (i, 0)),
            pl.BlockSpec((_RB, _OC), lambda i: (i, 0)),
        ],
        out_shape=[
            jax.ShapeDtypeStruct((_N, _D), jnp.float32),
            jax.ShapeDtypeStruct((_N, _OC), jnp.float32),
        ],
    )(p1, x, A, bias)

    tx1pk = lax.bitcast_convert_type(
        tx1.astype(jnp.bfloat16).reshape(_N, _D // 2, 2), jnp.float32)
    p2 = _make_sc2(Q)(tx1pk, src4, dst4, w_e, zrows)

    out_full = pl.pallas_call(
        _tc_combine2,
        grid=grid,
        in_specs=[
            pl.BlockSpec((_RB, _OC), lambda i: (i, 0)),
            pl.BlockSpec((_RB, _D), lambda i: (i, 0)),
            pl.BlockSpec((_NC, _RB, _D), lambda i: (0, i, 0)),
            pl.BlockSpec((_D, _OC), lambda i: (0, 0)),
            pl.BlockSpec((_D, _OC), lambda i: (0, 0)),
        ],
        out_specs=pl.BlockSpec((_RB, _OC), lambda i: (i, 0)),
        out_shape=jax.ShapeDtypeStruct((_N, _OC), jnp.float32),
    )(s_acc, tx1, p2, B, C)

    return out_full[:, :300]
